# baseline (speedup 1.0000x reference)
import functools

import jax
import jax.numpy as jnp
from jax import lax
from jax.experimental import pallas as pl
from jax.experimental.pallas import tpu as pltpu
from jax.experimental.pallas import tpu_sc as plsc

ROWS = 1024
COLS = 100000
LANES = 16
NUM_CORES = 2
NUM_SUBCORES = 16
NUM_WORKERS = NUM_CORES * NUM_SUBCORES
ROWS_PER_WORKER = ROWS // NUM_WORKERS
HALF = COLS // 2
NHALF = HALF // LANES
NACC = 1
NSTEP = NHALF // NACC
KAPPA = jnp.float32(1e30)
NEG_INF = jnp.float32(-jnp.inf)

_GATHER_DNUMS = lax.GatherDimensionNumbers(
    offset_dims=(), collapsed_slice_dims=(0,), start_index_map=(0,)
)


def _shuffle(v, idx):
    return lax.gather(
        v,
        idx.reshape(LANES, 1),
        _GATHER_DNUMS,
        slice_sizes=(1,),
        mode=lax.GatherScatterMode.PROMISE_IN_BOUNDS,
    )


def _butterfly(v, op, iota):
    for s in (8, 4, 2, 1):
        v = op(v, _shuffle(v, iota ^ s))
    return v


def _scan_half(buf, pairs):
    def step(j, pairs):
        base = j * (LANES * NACC)
        new = []
        for k, (m1, m2) in enumerate(pairs):
            v = buf[pl.ds(base + k * LANES, LANES)]
            t = jnp.minimum(m1, v)
            m1 = jnp.maximum(m1, v)
            m2 = jnp.maximum(m2, t)
            new.append((m1, m2))
        return tuple(new)

    return lax.fori_loop(0, NSTEP, step, pairs, unroll=10)


def _merge_pairs(pairs):
    (m1, m2), rest = pairs[0], pairs[1:]
    for n1, n2 in rest:
        t = jnp.minimum(m1, n1)
        m1 = jnp.maximum(m1, n1)
        m2 = jnp.maximum(jnp.maximum(m2, n2), t)
    return m1, m2


def _extract_block(buf, off_i, lane_i, iota):
    cv = buf[pl.ds(off_i, LANES)]
    return jnp.where(iota == lane_i, cv, NEG_INF)


def _margin_body(logits_hbm, labels_hbm, out_hbm, buf0, buf1, lab_buf, out_buf,
                 sem0, sem1):
    cid = lax.axis_index("c")
    sid = lax.axis_index("s")
    wid = sid * NUM_CORES + cid
    base = wid * ROWS_PER_WORKER

    pltpu.sync_copy(labels_hbm.at[pl.ds(base, ROWS_PER_WORKER)], lab_buf)

    iota = lax.iota(jnp.int32, LANES)
    iota_f = iota.astype(jnp.float32)

    pltpu.async_copy(logits_hbm.at[pl.ds(base * COLS, HALF)], buf0, sem0)

    def row_step(rl, out_carry):
        out0, out1 = out_carry
        r = base + rl

        lblk = (rl // LANES) * LANES
        labv = lab_buf[pl.ds(lblk, LANES)].astype(jnp.float32)
        label_fv = _butterfly(
            jnp.where(iota == rl - lblk, labv, jnp.float32(-1.0)),
            jnp.maximum,
            iota,
        )
        label_i = label_fv[0].astype(jnp.int32)

        init = tuple(
            (jnp.full((LANES,), NEG_INF, jnp.float32),
             jnp.full((LANES,), NEG_INF, jnp.float32))
            for _ in range(NACC)
        )

        pltpu.make_async_copy(
            logits_hbm.at[pl.ds(r * COLS, HALF)], buf0, sem0
        ).wait()
        pltpu.async_copy(logits_hbm.at[pl.ds(r * COLS + HALF, HALF)], buf1, sem1)
        pairs = _scan_half(buf0, init)
        b0 = jnp.minimum(jnp.maximum((label_i // LANES) * LANES, 0),
                         HALF - LANES)
        corr0v = _extract_block(buf0, b0, label_i - b0, iota)

        pltpu.make_async_copy(
            logits_hbm.at[pl.ds(r * COLS + HALF, HALF)], buf1, sem1
        ).wait()

        @pl.when(rl + 1 < ROWS_PER_WORKER)
        def _():
            pltpu.async_copy(
                logits_hbm.at[pl.ds((r + 1) * COLS, HALF)], buf0, sem0
            )

        pairs = _scan_half(buf1, pairs)
        l1 = label_i - HALF
        b1 = jnp.minimum(jnp.maximum((l1 // LANES) * LANES, 0), HALF - LANES)
        corr1v = _extract_block(buf1, b1, l1 - b1, iota)

        m1, m2 = _merge_pairs(pairs)
        row_topv = _butterfly(m1, jnp.maximum, iota)
        eq = m1 == row_topv
        cntv = _butterfly(jnp.where(eq, jnp.float32(1.0), jnp.float32(0.0)),
                          jnp.add, iota)
        m1_excl = jnp.where(eq, NEG_INF, m1)
        sec_m1 = _butterfly(m1_excl, jnp.maximum, iota)
        sec_m1 = jnp.where(cntv > 1.5, row_topv, sec_m1)
        row_secondv = jnp.maximum(sec_m1, _butterfly(m2, jnp.maximum, iota))

        corrv = jnp.where(label_fv < HALF, corr0v, corr1v)
        correctv = _butterfly(corrv, jnp.maximum, iota)

        max_incorrect = jnp.where(
            correctv == row_topv, row_secondv, row_topv
        )
        valv = jnp.minimum(max_incorrect - correctv, KAPPA)

        out0 = jnp.where(iota == rl, valv, out0)
        out1 = jnp.where(iota == rl - LANES, valv, out1)
        return out0, out1

    zeros = jnp.zeros((LANES,), jnp.float32)
    out0, out1 = lax.fori_loop(0, ROWS_PER_WORKER, row_step, (zeros, zeros))
    out_buf[pl.ds(0, LANES)] = out0
    out_buf[pl.ds(LANES, LANES)] = out1
    pltpu.sync_copy(out_buf, out_hbm.at[pl.ds(base, ROWS_PER_WORKER)])


@jax.jit
def _margin_loss(logits, labels):
    mesh = plsc.VectorSubcoreMesh(core_axis_name="c", subcore_axis_name="s")
    fn = functools.partial(
        pl.kernel,
        mesh=mesh,
        out_type=jax.ShapeDtypeStruct((ROWS,), jnp.float32),
        scratch_types=[
            pltpu.VMEM((HALF,), jnp.float32),
            pltpu.VMEM((HALF,), jnp.float32),
            pltpu.VMEM((ROWS_PER_WORKER,), jnp.int32),
            pltpu.VMEM((ROWS_PER_WORKER,), jnp.float32),
            pltpu.SemaphoreType.DMA,
            pltpu.SemaphoreType.DMA,
        ],
    )(_margin_body)
    return fn(logits.reshape(-1), labels)


def kernel(logits, labels):
    return _margin_loss(logits, labels.astype(jnp.int32))


# speedup vs baseline: 1.6562x; 1.6562x over previous
import functools

import jax
import jax.numpy as jnp
from jax import lax
from jax.experimental import pallas as pl
from jax.experimental.pallas import tpu as pltpu
from jax.experimental.pallas import tpu_sc as plsc

ROWS = 1024
COLS = 100000
LANES = 16
NUM_CORES = 2
NUM_SUBCORES = 16
NUM_WORKERS = NUM_CORES * NUM_SUBCORES
ROWS_PER_WORKER = ROWS // NUM_WORKERS
NVREG = COLS // LANES
NACC = 2
NSTEP = NVREG // NACC
KAPPA = 1e30
NEG_INF = float("-inf")

_GATHER_DNUMS = lax.GatherDimensionNumbers(
    offset_dims=(), collapsed_slice_dims=(0,), start_index_map=(0,)
)


def _shuffle(v, idx):
    return lax.gather(
        v,
        idx.reshape(LANES, 1),
        _GATHER_DNUMS,
        slice_sizes=(1,),
        mode=lax.GatherScatterMode.PROMISE_IN_BOUNDS,
    )


def _butterfly(v, op, iota):
    for s in (8, 4, 2, 1):
        v = op(v, _shuffle(v, iota ^ s))
    return v


def _scan_row(buf, pairs):
    def step(j, pairs):
        base = j * (LANES * NACC)
        new = []
        for k, (m1, m2) in enumerate(pairs):
            v = buf[pl.ds(base + k * LANES, LANES)]
            t = jnp.minimum(m1, v)
            m1 = jnp.maximum(m1, v)
            m2 = jnp.maximum(m2, t)
            new.append((m1, m2))
        return tuple(new)

    return lax.fori_loop(0, NSTEP, step, pairs, unroll=5)


def _merge_pairs(pairs):
    (m1, m2), rest = pairs[0], pairs[1:]
    for n1, n2 in rest:
        t = jnp.minimum(m1, n1)
        m1 = jnp.maximum(m1, n1)
        m2 = jnp.maximum(jnp.maximum(m2, n2), t)
    return m1, m2


def _margin_body(logits_hbm, labels_hbm, out_hbm, row_buf, lab_buf, out_buf):
    cid = lax.axis_index("c")
    sid = lax.axis_index("s")
    wid = sid * NUM_CORES + cid
    base = wid * ROWS_PER_WORKER

    pltpu.sync_copy(labels_hbm.at[pl.ds(base, ROWS_PER_WORKER)], lab_buf)

    iota = lax.iota(jnp.int32, LANES)

    def row_step(rl, out_carry):
        out0, out1 = out_carry
        r = base + rl
        pltpu.sync_copy(logits_hbm.at[r], row_buf)

        lblk = (rl // LANES) * LANES
        labv = lab_buf[pl.ds(lblk, LANES)].astype(jnp.float32)
        label_fv = _butterfly(
            jnp.where(iota == rl - lblk, labv, jnp.float32(-1.0)),
            jnp.maximum,
            iota,
        )
        label_i = label_fv[0].astype(jnp.int32)

        init = tuple(
            (jnp.full((LANES,), NEG_INF, jnp.float32),
             jnp.full((LANES,), NEG_INF, jnp.float32))
            for _ in range(NACC)
        )
        pairs = _scan_row(row_buf, init)

        cblk = (label_i // LANES) * LANES
        cv = row_buf[pl.ds(cblk, LANES)]
        correctv = _butterfly(
            jnp.where(iota == label_i - cblk, cv, NEG_INF), jnp.maximum, iota
        )

        m1, m2 = _merge_pairs(pairs)
        row_topv = _butterfly(m1, jnp.maximum, iota)
        eq = m1 == row_topv
        cntv = _butterfly(jnp.where(eq, jnp.float32(1.0), jnp.float32(0.0)),
                          jnp.add, iota)
        m1_excl = jnp.where(eq, NEG_INF, m1)
        sec_m1 = _butterfly(m1_excl, jnp.maximum, iota)
        sec_m1 = jnp.where(cntv > 1.5, row_topv, sec_m1)
        row_secondv = jnp.maximum(sec_m1, _butterfly(m2, jnp.maximum, iota))

        max_incorrect = jnp.where(correctv == row_topv, row_secondv, row_topv)
        valv = jnp.minimum(max_incorrect - correctv, KAPPA)

        out0 = jnp.where(iota == rl, valv, out0)
        out1 = jnp.where(iota == rl - LANES, valv, out1)
        return out0, out1

    zeros = jnp.zeros((LANES,), jnp.float32)
    out0, out1 = lax.fori_loop(0, ROWS_PER_WORKER, row_step, (zeros, zeros))
    out_buf[pl.ds(0, LANES)] = out0
    out_buf[pl.ds(LANES, LANES)] = out1
    pltpu.sync_copy(out_buf, out_hbm.at[pl.ds(base, ROWS_PER_WORKER)])


@jax.jit
def _margin_loss(logits, labels):
    mesh = plsc.VectorSubcoreMesh(core_axis_name="c", subcore_axis_name="s")
    fn = functools.partial(
        pl.kernel,
        mesh=mesh,
        out_type=jax.ShapeDtypeStruct((ROWS,), jnp.float32),
        scratch_types=[
            pltpu.VMEM((COLS,), jnp.float32),
            pltpu.VMEM((ROWS_PER_WORKER,), jnp.int32),
            pltpu.VMEM((ROWS_PER_WORKER,), jnp.float32),
        ],
    )(_margin_body)
    return fn(logits, labels)


def kernel(logits, labels):
    return _margin_loss(logits, labels.astype(jnp.int32))


# speedup vs baseline: 1.9250x; 1.1623x over previous
import functools

import jax
import jax.numpy as jnp
from jax import lax
from jax.experimental import pallas as pl
from jax.experimental.pallas import tpu as pltpu
from jax.experimental.pallas import tpu_sc as plsc

ROWS = 1024
COLS = 100000
LANES = 16
NUM_CORES = 2
NUM_SUBCORES = 16
NUM_WORKERS = NUM_CORES * NUM_SUBCORES
ROWS_PER_WORKER = ROWS // NUM_WORKERS
NACC = 2
UNROLL = 5
GRAIN = LANES * NACC
C0 = 49920
C1 = COLS - C0
NSTEP0 = C0 // GRAIN
NSTEP1 = C1 // GRAIN
KAPPA = 1e30
NEG_INF = float("-inf")

_GATHER_DNUMS = lax.GatherDimensionNumbers(
    offset_dims=(), collapsed_slice_dims=(0,), start_index_map=(0,)
)


def _shuffle(v, idx):
    return lax.gather(
        v,
        idx.reshape(LANES, 1),
        _GATHER_DNUMS,
        slice_sizes=(1,),
        mode=lax.GatherScatterMode.PROMISE_IN_BOUNDS,
    )


def _butterfly(v, op, iota):
    for s in (8, 4, 2, 1):
        v = op(v, _shuffle(v, iota ^ s))
    return v


def _scan_chunk(buf, pairs, nstep):
    def step(j, pairs):
        base = j * GRAIN
        new = []
        for k, (m1, m2) in enumerate(pairs):
            v = buf[pl.ds(base + k * LANES, LANES)]
            t = jnp.minimum(m1, v)
            m1 = jnp.maximum(m1, v)
            m2 = jnp.maximum(m2, t)
            new.append((m1, m2))
        return tuple(new)

    return lax.fori_loop(0, nstep, step, pairs, unroll=UNROLL)


def _merge_pairs(pairs):
    (m1, m2), rest = pairs[0], pairs[1:]
    for n1, n2 in rest:
        t = jnp.minimum(m1, n1)
        m1 = jnp.maximum(m1, n1)
        m2 = jnp.maximum(jnp.maximum(m2, n2), t)
    return m1, m2


def _extract(buf, pos_i, limit, iota):
    blk = jnp.minimum(jnp.maximum((pos_i // LANES) * LANES, 0), limit - LANES)
    cv = buf[pl.ds(blk, LANES)]
    return jnp.where(iota == pos_i - blk, cv, NEG_INF)


def _margin_body(logits_hbm, labels_hbm, out_hbm, buf0, buf1, lab_buf, out_buf,
                 sem0, sem1):
    cid = lax.axis_index("c")
    sid = lax.axis_index("s")
    wid = sid * NUM_CORES + cid
    base = wid * ROWS_PER_WORKER

    pltpu.sync_copy(labels_hbm.at[pl.ds(base, ROWS_PER_WORKER)], lab_buf)

    iota = lax.iota(jnp.int32, LANES)

    pltpu.async_copy(logits_hbm.at[base].at[pl.ds(0, C0)], buf0, sem0)

    def row_step(rl, out_carry):
        out0, out1 = out_carry
        r = base + rl

        lblk = (rl // LANES) * LANES
        labv = lab_buf[pl.ds(lblk, LANES)].astype(jnp.float32)
        label_fv = _butterfly(
            jnp.where(iota == rl - lblk, labv, jnp.float32(-1.0)),
            jnp.maximum,
            iota,
        )
        label_i = label_fv[0].astype(jnp.int32)

        init = tuple(
            (jnp.full((LANES,), NEG_INF, jnp.float32),
             jnp.full((LANES,), NEG_INF, jnp.float32))
            for _ in range(NACC)
        )

        pltpu.make_async_copy(
            logits_hbm.at[r].at[pl.ds(0, C0)], buf0, sem0
        ).wait()
        pltpu.async_copy(logits_hbm.at[r].at[pl.ds(C0, C1)], buf1, sem1)
        pairs = _scan_chunk(buf0, init, NSTEP0)
        corr0v = _extract(buf0, label_i, C0, iota)

        pltpu.make_async_copy(
            logits_hbm.at[r].at[pl.ds(C0, C1)], buf1, sem1
        ).wait()

        @pl.when(rl + 1 < ROWS_PER_WORKER)
        def _():
            pltpu.async_copy(
                logits_hbm.at[r + 1].at[pl.ds(0, C0)], buf0, sem0
            )

        pairs = _scan_chunk(buf1, pairs, NSTEP1)
        corr1v = _extract(buf1, label_i - C0, C1, iota)

        m1, m2 = _merge_pairs(pairs)
        row_topv = _butterfly(m1, jnp.maximum, iota)
        eq = m1 == row_topv
        cntv = _butterfly(jnp.where(eq, jnp.float32(1.0), jnp.float32(0.0)),
                          jnp.add, iota)
        m1_excl = jnp.where(eq, NEG_INF, m1)
        sec_m1 = _butterfly(m1_excl, jnp.maximum, iota)
        sec_m1 = jnp.where(cntv > 1.5, row_topv, sec_m1)
        row_secondv = jnp.maximum(sec_m1, _butterfly(m2, jnp.maximum, iota))

        corrv = jnp.where(label_fv < C0, corr0v, corr1v)
        correctv = _butterfly(corrv, jnp.maximum, iota)

        max_incorrect = jnp.where(correctv == row_topv, row_secondv, row_topv)
        valv = jnp.minimum(max_incorrect - correctv, KAPPA)

        out0 = jnp.where(iota == rl, valv, out0)
        out1 = jnp.where(iota == rl - LANES, valv, out1)
        return out0, out1

    zeros = jnp.zeros((LANES,), jnp.float32)
    out0, out1 = lax.fori_loop(0, ROWS_PER_WORKER, row_step, (zeros, zeros))
    out_buf[pl.ds(0, LANES)] = out0
    out_buf[pl.ds(LANES, LANES)] = out1
    pltpu.sync_copy(out_buf, out_hbm.at[pl.ds(base, ROWS_PER_WORKER)])


@jax.jit
def _margin_loss(logits, labels):
    mesh = plsc.VectorSubcoreMesh(core_axis_name="c", subcore_axis_name="s")
    fn = functools.partial(
        pl.kernel,
        mesh=mesh,
        out_type=jax.ShapeDtypeStruct((ROWS,), jnp.float32),
        scratch_types=[
            pltpu.VMEM((C0,), jnp.float32),
            pltpu.VMEM((C1,), jnp.float32),
            pltpu.VMEM((ROWS_PER_WORKER,), jnp.int32),
            pltpu.VMEM((ROWS_PER_WORKER,), jnp.float32),
            pltpu.SemaphoreType.DMA,
            pltpu.SemaphoreType.DMA,
        ],
    )(_margin_body)
    return fn(logits, labels)


def kernel(logits, labels):
    return _margin_loss(logits, labels.astype(jnp.int32))
